# baseline (speedup 1.0000x reference)
import jax
import jax.numpy as jnp
from jax import lax
from jax.experimental import pallas as pl
from jax.experimental.pallas import tpu as pltpu
from jax.experimental.pallas import tpu_sc as plsc

H = 128
NPROD = 10000
NCUST = 10000
N_LBL = 100000

NC = 2
NS = 16
NW = NC * NS
CHUNK = 64
CW = 16
NPAD = 10240
SLABS = 8
SLABCH = 20
E_PAD = NW * SLABS * SLABCH * CHUNK
CNT_OFF = 1256
GCHUNK = 128
HALF = 102400

_MESH = plsc.VectorSubcoreMesh(core_axis_name="c", subcore_axis_name="s")


def _make_segsum(slabs):
  rows_sub = NPAD // NS

  scratch = dict(
      idx_s=pltpu.VMEM((SLABCH, CHUNK), jnp.int32),
      idx_d=pltpu.VMEM((SLABCH, CHUNK), jnp.int32),
      rows=pltpu.VMEM((CHUNK, H), jnp.float32),
      izb=pltpu.VMEM((NPAD // NS // CHUNK, CHUNK), jnp.int32),
      acc_sh=pltpu.VMEM_SHARED((NPAD, H), jnp.float32),
      sem=pltpu.SemaphoreType.DMA,
  )
  n_rep = rows_sub // CHUNK

  def body(x_hbm, src_hbm, dst_hbm, zrow_hbm, iota_hbm, acc_out,
           *, idx_s, idx_d, rows, izb, acc_sh, sem):
    cid = lax.axis_index("c")
    sid = lax.axis_index("s")
    wid = cid * NS + sid

    pltpu.sync_copy(iota_hbm.at[sid], izb)
    pltpu.sync_copy(zrow_hbm, rows)
    for r in range(n_rep):
      pltpu.sync_copy(rows, acc_sh.at[izb.at[r]])
    plsc.subcore_barrier()

    def slab(s, carry):
      pltpu.sync_copy(src_hbm.at[wid * slabs + s], idx_s)
      pltpu.sync_copy(dst_hbm.at[wid * slabs + s], idx_d)

      def step(c, carry2):
        pltpu.async_copy(x_hbm.at[idx_s.at[c]], rows, sem).wait()
        pltpu.sync_copy(rows, acc_sh.at[idx_d.at[c]], add=True)
        return carry2
      lax.fori_loop(0, SLABCH, step, 0)
      return carry
    lax.fori_loop(0, slabs, slab, 0)

    plsc.subcore_barrier()
    for r in range(n_rep):
      out_sl = pl.ds(cid * NPAD + sid * rows_sub + r * CHUNK, CHUNK)
      pltpu.async_copy(acc_sh.at[izb.at[r]], rows, sem).wait()
      pltpu.sync_copy(rows, acc_out.at[out_sl])

  return pl.kernel(
      body,
      out_type=jax.ShapeDtypeStruct((NC * NPAD, H), jnp.float32),
      mesh=_MESH,
      scratch_types=scratch,
  )


def _make_gather(n_out, nchunks):
  scratch = dict(
      idx_v=pltpu.VMEM((nchunks, GCHUNK), jnp.int32),
      rows=pltpu.VMEM((GCHUNK, H), jnp.float32),
      sem=pltpu.SemaphoreType.DMA,
  )

  def body(table_hbm, idx_hbm, out_hbm, *, idx_v, rows, sem):
    cid = lax.axis_index("c")
    sid = lax.axis_index("s")
    wid = cid * NS + sid
    base = wid * nchunks * GCHUNK
    pltpu.sync_copy(idx_hbm.at[wid], idx_v)

    def step(c, carry):
      pltpu.async_copy(table_hbm.at[idx_v.at[c]], rows, sem).wait()
      pltpu.sync_copy(rows, out_hbm.at[pl.ds(base + c * GCHUNK, GCHUNK)])
      return carry
    lax.fori_loop(0, nchunks, step, 0)

  return pl.kernel(
      body,
      out_type=jax.ShapeDtypeStruct((n_out, H), jnp.float32),
      mesh=_MESH,
      scratch_types=scratch,
  )


def _dot_t(a, w):
  return lax.dot_general(a, w, (((1,), (1,)), ((), ())),
                         preferred_element_type=jnp.float32)


def _sage_combine(acc, cnt, x_dst, wl, wr, b):
  n = x_dst.shape[0]
  bs = 1000
  acc = acc.reshape(NC, NPAD, H)

  def body(a_ref, c_ref, xd_ref, wl_ref, wr_ref, b_ref, o_ref):
    p = a_ref[0] + a_ref[1]
    c = c_ref[0, :, 0:1] + c_ref[1, :, 0:1]
    agg = p / jnp.maximum(c, 1.0)
    y = _dot_t(agg, wl_ref[...]) + b_ref[...] + _dot_t(xd_ref[...], wr_ref[...])
    o_ref[...] = jnp.maximum(y, 0.0)

  return pl.pallas_call(
      body,
      grid=(n // bs,),
      in_specs=[
          pl.BlockSpec((NC, bs, H), lambda i: (0, i, 0)),
          pl.BlockSpec((NC, bs, CW), lambda i: (0, i, 0)),
          pl.BlockSpec((bs, H), lambda i: (i, 0)),
          pl.BlockSpec((H, H), lambda i: (0, 0)),
          pl.BlockSpec((H, H), lambda i: (0, 0)),
          pl.BlockSpec((1, H), lambda i: (0, 0)),
      ],
      out_specs=pl.BlockSpec((bs, H), lambda i: (i, 0)),
      out_shape=jax.ShapeDtypeStruct((n, H), jnp.float32),
  )(acc, cnt, x_dst, wl, wr, b)


def _linear(x, w, b):
  n = x.shape[0]
  o = w.shape[0]
  bs = 1000

  def body(x_ref, w_ref, b_ref, o_ref):
    o_ref[...] = _dot_t(x_ref[...], w_ref[...]) + b_ref[...]

  return pl.pallas_call(
      body,
      grid=(n // bs,),
      in_specs=[
          pl.BlockSpec((bs, H), lambda i: (i, 0)),
          pl.BlockSpec((o, H), lambda i: (0, 0)),
          pl.BlockSpec((1, o), lambda i: (0, 0)),
      ],
      out_specs=pl.BlockSpec((bs, o), lambda i: (i, 0)),
      out_shape=jax.ShapeDtypeStruct((n, o), jnp.float32),
  )(x, w, b)


def _decoder_final(g, w2, b2):
  bs = 800

  def body(gc_ref, gp_ref, w2_ref, b2_ref, o_ref):
    h = jnp.maximum(gc_ref[...] + gp_ref[...], 0.0)
    y = lax.dot_general(h, w2_ref[...], (((1,), (0,)), ((), ())),
                        preferred_element_type=jnp.float32)
    o_ref[...] = y[:, 0:1] + b2_ref[0, 0]

  return pl.pallas_call(
      body,
      grid=(N_LBL // bs,),
      in_specs=[
          pl.BlockSpec((bs, H), lambda i: (i, 0)),
          pl.BlockSpec((bs, H), lambda i: (i + HALF // bs, 0)),
          pl.BlockSpec((H, H), lambda i: (0, 0)),
          pl.BlockSpec((1, 1), lambda i: (0, 0)),
      ],
      out_specs=pl.BlockSpec((bs, 1), lambda i: (i, 0)),
      out_shape=jax.ShapeDtypeStruct((N_LBL, 1), jnp.float32),
  )(g, g, jnp.broadcast_to(w2.T, (H, H)), b2)


_GN_CH = 2 * HALF // NW // GCHUNK

_segsum = _make_segsum(SLABS)
_segsum2 = _make_segsum(2 * SLABS)
_gather = _make_gather(2 * HALF, _GN_CH)


def _pad_flat(ei, n_dst):
  e = ei.shape[1]
  src = jnp.concatenate([ei[0], jnp.zeros((E_PAD - e,), jnp.int32)])
  dst = jnp.concatenate([ei[1], jnp.full((E_PAD - e,), n_dst, jnp.int32)])
  return src, dst


def _slabbed(a, slabs):
  return a.reshape(NW * slabs, SLABCH, CHUNK)


@jax.jit
def kernel(x_product, x_customer, edge_index_pp, edge_index_pc,
           edge_label_index, Wl_i1, Wr_i1, Wl_i2, Wr_i2, W_ilin, Wl_u1, Wr_u1,
           Wl_u2, Wr_u2, Wl_u3, Wr_u3, W_ulin, W_d1, W_d2, bl_i1, bl_i2,
           b_ilin, bl_u1, bl_u2, bl_u3, b_ulin, b_d1, b_d2):
  src_pp, dst_pp = _pad_flat(edge_index_pp, NPROD)
  src_pc, dst_pc = _pad_flat(edge_index_pc, NCUST)

  zrow = jnp.zeros((CHUNK, H), jnp.float32)
  iota = jnp.arange(NPAD, dtype=jnp.int32).reshape(
      NS, NPAD // NS // CHUNK, CHUNK)

  acc_pp = _segsum(x_product, _slabbed(src_pp, SLABS),
                   _slabbed(dst_pp, SLABS), zrow, iota)
  acc_pcx = _segsum(x_product, _slabbed(src_pc, SLABS),
                    _slabbed(dst_pc, SLABS), zrow, iota)

  ones8 = (jnp.arange(H, dtype=jnp.int32)[None, :] // CW
           == jnp.arange(8, dtype=jnp.int32)[:, None]).astype(jnp.float32)
  csrc = jnp.concatenate([dst_pp % 8, dst_pc % 8])
  cdst = jnp.concatenate([dst_pp // 8, CNT_OFF + dst_pc // 8])
  co = _segsum2(ones8, _slabbed(csrc, 2 * SLABS),
                _slabbed(cdst, 2 * SLABS), zrow, iota)
  co = co.reshape(NC, NPAD, 8, CW)
  cnt_pp = co[:, :NPROD // 8].reshape(NC, NPROD, CW)
  cnt_pc = co[:, CNT_OFF:CNT_OFF + NCUST // 8].reshape(NC, NCUST, CW)

  b2d = lambda b: b.reshape(1, -1)
  px = _sage_combine(acc_pp, cnt_pp, x_product, Wl_u1, Wr_u1, b2d(bl_u1))
  ix1 = _sage_combine(acc_pp, cnt_pp, x_product, Wl_i1, Wr_i1, b2d(bl_i1))
  cx1 = _sage_combine(acc_pcx, cnt_pc, x_customer, Wl_u2, Wr_u2, b2d(bl_u2))

  acc_pcpx = _segsum(px, _slabbed(src_pc, SLABS),
                     _slabbed(dst_pc, SLABS), zrow, iota)
  acc_ppix = _segsum(ix1, _slabbed(src_pp, SLABS),
                     _slabbed(dst_pp, SLABS), zrow, iota)

  cx2 = _sage_combine(acc_pcpx, cnt_pc, cx1, Wl_u3, Wr_u3, b2d(bl_u3))
  ix2 = _sage_combine(acc_ppix, cnt_pp, ix1, Wl_i2, Wr_i2, b2d(bl_i2))
  zc = _linear(cx2, W_ulin, b2d(b_ulin))
  zp = _linear(ix2, W_ilin, b2d(b_ilin))
  uc = _linear(zc, W_d1[:, :H], b2d(b_d1))
  up = _linear(zp, W_d1[:, H:], jnp.zeros((1, H), jnp.float32))

  table = jnp.concatenate([uc, up], axis=0)
  pad = jnp.zeros((HALF - N_LBL,), jnp.int32)
  gidx = jnp.concatenate(
      [edge_label_index[0], pad, edge_label_index[1] + NCUST, pad], axis=0)
  g = _gather(table, gidx.reshape(NW, _GN_CH, GCHUNK))

  return _decoder_final(g, W_d2, b2d(b_d2))


# speedup vs baseline: 2.0155x; 2.0155x over previous
import jax
import jax.numpy as jnp
from jax import lax
from jax.experimental import pallas as pl
from jax.experimental.pallas import tpu as pltpu
from jax.experimental.pallas import tpu_sc as plsc

H = 128
NPROD = 10000
NCUST = 10000
N_LBL = 100000

NC = 2
NS = 16
NW = NC * NS
CHUNK = 64
CW = 16
NPAD = 10240
SLABS = 8
SLABCH = 20
E_PAD = NW * SLABS * SLABCH * CHUNK
CNT_OFF = 1256
GCHUNK = 128
HALF = 102400

_MESH = plsc.VectorSubcoreMesh(core_axis_name="c", subcore_axis_name="s")


def _make_segsum(slabs):
  rows_sub = NPAD // NS

  scratch = dict(
      idx_s=pltpu.VMEM((SLABCH, CHUNK), jnp.int32),
      idx_d=pltpu.VMEM((SLABCH, CHUNK), jnp.int32),
      rows=pltpu.VMEM((CHUNK, H), jnp.float32),
      izb=pltpu.VMEM((NPAD // NS // CHUNK, CHUNK), jnp.int32),
      acc_sh=pltpu.VMEM_SHARED((NPAD, H), jnp.float32),
      sem=pltpu.SemaphoreType.DMA,
  )
  n_rep = rows_sub // CHUNK

  def body(x_hbm, src_hbm, dst_hbm, zrow_hbm, iota_hbm, acc_out,
           *, idx_s, idx_d, rows, izb, acc_sh, sem):
    cid = lax.axis_index("c")
    sid = lax.axis_index("s")
    wid = cid * NS + sid

    pltpu.sync_copy(iota_hbm.at[sid], izb)
    pltpu.sync_copy(zrow_hbm, rows)
    for r in range(n_rep):
      pltpu.sync_copy(rows, acc_sh.at[izb.at[r]])
    plsc.subcore_barrier()

    def slab(s, carry):
      pltpu.sync_copy(src_hbm.at[wid * slabs + s], idx_s)
      pltpu.sync_copy(dst_hbm.at[wid * slabs + s], idx_d)

      def step(c, carry2):
        pltpu.async_copy(x_hbm.at[idx_s.at[c]], rows, sem).wait()
        pltpu.sync_copy(rows, acc_sh.at[idx_d.at[c]], add=True)
        return carry2
      lax.fori_loop(0, SLABCH, step, 0)
      return carry
    lax.fori_loop(0, slabs, slab, 0)

    plsc.subcore_barrier()
    for r in range(n_rep):
      out_sl = pl.ds(cid * NPAD + sid * rows_sub + r * CHUNK, CHUNK)
      pltpu.async_copy(acc_sh.at[izb.at[r]], rows, sem).wait()
      pltpu.sync_copy(rows, acc_out.at[out_sl])

  return pl.kernel(
      body,
      out_type=jax.ShapeDtypeStruct((NC * NPAD, H), jnp.float32),
      mesh=_MESH,
      scratch_types=scratch,
  )


def _make_gather(n_out, nchunks):
  scratch = dict(
      idx_v=pltpu.VMEM((nchunks, GCHUNK), jnp.int32),
      rows=pltpu.VMEM((GCHUNK, H), jnp.float32),
      sem=pltpu.SemaphoreType.DMA,
  )

  def body(table_hbm, idx_hbm, out_hbm, *, idx_v, rows, sem):
    cid = lax.axis_index("c")
    sid = lax.axis_index("s")
    wid = cid * NS + sid
    base = wid * nchunks * GCHUNK
    pltpu.sync_copy(idx_hbm.at[wid], idx_v)

    def step(c, carry):
      pltpu.async_copy(table_hbm.at[idx_v.at[c]], rows, sem).wait()
      pltpu.sync_copy(rows, out_hbm.at[pl.ds(base + c * GCHUNK, GCHUNK)])
      return carry
    lax.fori_loop(0, nchunks, step, 0)

  return pl.kernel(
      body,
      out_type=jax.ShapeDtypeStruct((n_out, H), jnp.float32),
      mesh=_MESH,
      scratch_types=scratch,
  )


def _dot_t(a, w):
  return lax.dot_general(a, w, (((1,), (1,)), ((), ())),
                         preferred_element_type=jnp.float32)


def _sage_combine(acc, cnt, x_dst, wl, wr, b):
  n = x_dst.shape[0]
  bs = 1000
  acc = acc.reshape(NC, NPAD, H)

  def body(a_ref, c_ref, xd_ref, wl_ref, wr_ref, b_ref, o_ref):
    p = a_ref[0] + a_ref[1]
    c = c_ref[0, :, 0:1] + c_ref[1, :, 0:1]
    agg = p / jnp.maximum(c, 1.0)
    y = _dot_t(agg, wl_ref[...]) + b_ref[...] + _dot_t(xd_ref[...], wr_ref[...])
    o_ref[...] = jnp.maximum(y, 0.0)

  return pl.pallas_call(
      body,
      grid=(n // bs,),
      in_specs=[
          pl.BlockSpec((NC, bs, H), lambda i: (0, i, 0)),
          pl.BlockSpec((NC, bs, CW), lambda i: (0, i, 0)),
          pl.BlockSpec((bs, H), lambda i: (i, 0)),
          pl.BlockSpec((H, H), lambda i: (0, 0)),
          pl.BlockSpec((H, H), lambda i: (0, 0)),
          pl.BlockSpec((1, H), lambda i: (0, 0)),
      ],
      out_specs=pl.BlockSpec((bs, H), lambda i: (i, 0)),
      out_shape=jax.ShapeDtypeStruct((n, H), jnp.float32),
  )(acc, cnt, x_dst, wl, wr, b)


def _linear(x, w, b):
  n = x.shape[0]
  o = w.shape[0]
  bs = 1000

  def body(x_ref, w_ref, b_ref, o_ref):
    o_ref[...] = _dot_t(x_ref[...], w_ref[...]) + b_ref[...]

  return pl.pallas_call(
      body,
      grid=(n // bs,),
      in_specs=[
          pl.BlockSpec((bs, H), lambda i: (i, 0)),
          pl.BlockSpec((o, H), lambda i: (0, 0)),
          pl.BlockSpec((1, o), lambda i: (0, 0)),
      ],
      out_specs=pl.BlockSpec((bs, o), lambda i: (i, 0)),
      out_shape=jax.ShapeDtypeStruct((n, o), jnp.float32),
  )(x, w, b)


def _decoder_final(g, w2, b2):
  bs = 800

  def body(gc_ref, gp_ref, w2_ref, b2_ref, o_ref):
    h = jnp.maximum(gc_ref[...] + gp_ref[...], 0.0)
    y = lax.dot_general(h, w2_ref[...], (((1,), (0,)), ((), ())),
                        preferred_element_type=jnp.float32)
    o_ref[...] = y[:, 0:1] + b2_ref[0, 0]

  return pl.pallas_call(
      body,
      grid=(N_LBL // bs,),
      in_specs=[
          pl.BlockSpec((bs, H), lambda i: (i, 0)),
          pl.BlockSpec((bs, H), lambda i: (i + HALF // bs, 0)),
          pl.BlockSpec((H, H), lambda i: (0, 0)),
          pl.BlockSpec((1, 1), lambda i: (0, 0)),
      ],
      out_specs=pl.BlockSpec((bs, 1), lambda i: (i, 0)),
      out_shape=jax.ShapeDtypeStruct((N_LBL, 1), jnp.float32),
  )(g, g, jnp.broadcast_to(w2.T, (H, H)), b2)


_GN_CH = 2 * HALF // NW // GCHUNK

_segsum = _make_segsum(SLABS)
_segsum2 = _make_segsum(2 * SLABS)
_gather = _make_gather(2 * HALF, _GN_CH)


def _pad_flat(ei, n_dst):
  e = ei.shape[1]
  src = jnp.concatenate([ei[0], jnp.zeros((E_PAD - e,), jnp.int32)])
  dst = jnp.concatenate([ei[1], jnp.full((E_PAD - e,), n_dst, jnp.int32)])
  return src, dst


def _slabbed(a, slabs):
  return a.reshape(NW * slabs, SLABCH, CHUNK)


@jax.jit
def kernel(x_product, x_customer, edge_index_pp, edge_index_pc,
           edge_label_index, Wl_i1, Wr_i1, Wl_i2, Wr_i2, W_ilin, Wl_u1, Wr_u1,
           Wl_u2, Wr_u2, Wl_u3, Wr_u3, W_ulin, W_d1, W_d2, bl_i1, bl_i2,
           b_ilin, bl_u1, bl_u2, bl_u3, b_ulin, b_d1, b_d2):
  src_pp, dst_pp = _pad_flat(edge_index_pp, NPROD)
  src_pc, dst_pc = _pad_flat(edge_index_pc, NCUST)

  zrow = jnp.zeros((CHUNK, H), jnp.float32)
  iota = jnp.arange(NPAD, dtype=jnp.int32).reshape(
      NS, NPAD // NS // CHUNK, CHUNK)

  acc_pp = _segsum(x_product, _slabbed(src_pp, SLABS),
                   _slabbed(dst_pp, SLABS), zrow, iota)
  acc_pcx = _segsum(x_product, _slabbed(src_pc, SLABS),
                    _slabbed(dst_pc, SLABS), zrow, iota)

  ones8 = (jnp.arange(H, dtype=jnp.int32)[None, :] // CW
           == jnp.arange(8, dtype=jnp.int32)[:, None]).astype(jnp.float32)
  rep = 1024
  ones8 = jnp.tile(ones8, (rep, 1))
  spread = 8 * (jnp.arange(2 * E_PAD, dtype=jnp.int32) % rep)
  csrc = jnp.concatenate([dst_pp % 8, dst_pc % 8]) + spread
  cdst = jnp.concatenate([dst_pp // 8, CNT_OFF + dst_pc // 8])
  co = _segsum2(ones8, _slabbed(csrc, 2 * SLABS),
                _slabbed(cdst, 2 * SLABS), zrow, iota)
  co = co.reshape(NC, NPAD, 8, CW)
  cnt_pp = co[:, :NPROD // 8].reshape(NC, NPROD, CW)
  cnt_pc = co[:, CNT_OFF:CNT_OFF + NCUST // 8].reshape(NC, NCUST, CW)

  b2d = lambda b: b.reshape(1, -1)
  px = _sage_combine(acc_pp, cnt_pp, x_product, Wl_u1, Wr_u1, b2d(bl_u1))
  ix1 = _sage_combine(acc_pp, cnt_pp, x_product, Wl_i1, Wr_i1, b2d(bl_i1))
  cx1 = _sage_combine(acc_pcx, cnt_pc, x_customer, Wl_u2, Wr_u2, b2d(bl_u2))

  acc_pcpx = _segsum(px, _slabbed(src_pc, SLABS),
                     _slabbed(dst_pc, SLABS), zrow, iota)
  acc_ppix = _segsum(ix1, _slabbed(src_pp, SLABS),
                     _slabbed(dst_pp, SLABS), zrow, iota)

  cx2 = _sage_combine(acc_pcpx, cnt_pc, cx1, Wl_u3, Wr_u3, b2d(bl_u3))
  ix2 = _sage_combine(acc_ppix, cnt_pp, ix1, Wl_i2, Wr_i2, b2d(bl_i2))
  zc = _linear(cx2, W_ulin, b2d(b_ulin))
  zp = _linear(ix2, W_ilin, b2d(b_ilin))
  uc = _linear(zc, W_d1[:, :H], b2d(b_d1))
  up = _linear(zp, W_d1[:, H:], jnp.zeros((1, H), jnp.float32))

  table = jnp.concatenate([uc, up], axis=0)
  pad = jnp.zeros((HALF - N_LBL,), jnp.int32)
  gidx = jnp.concatenate(
      [edge_label_index[0], pad, edge_label_index[1] + NCUST, pad], axis=0)
  g = _gather(table, gidx.reshape(NW, _GN_CH, GCHUNK))

  return _decoder_final(g, W_d2, b2d(b_d2))


# speedup vs baseline: 2.2312x; 1.1070x over previous
import jax
import jax.numpy as jnp
from jax import lax
from jax.experimental import pallas as pl
from jax.experimental.pallas import tpu as pltpu
from jax.experimental.pallas import tpu_sc as plsc

H = 128
NPROD = 10000
NCUST = 10000
N_LBL = 100000

NC = 2
NS = 16
NW = NC * NS
CHUNK = 64
CW = 16
NPAD = 10240
SLABS = 8
SLABCH = 20
E_PAD = NW * SLABS * SLABCH * CHUNK
CNT_OFF = 1256
GCHUNK = 128
HALF = 102400

_MESH = plsc.VectorSubcoreMesh(core_axis_name="c", subcore_axis_name="s")


def _make_segsum(slabs):
  rows_sub = NPAD // NS

  scratch = dict(
      idx_s=pltpu.VMEM((SLABCH, CHUNK), jnp.int32),
      idx_d=pltpu.VMEM((SLABCH, CHUNK), jnp.int32),
      rows=pltpu.VMEM((CHUNK, H), jnp.float32),
      rows2=pltpu.VMEM((CHUNK, H), jnp.float32),
      izb=pltpu.VMEM((NPAD // NS // CHUNK, CHUNK), jnp.int32),
      acc_sh=pltpu.VMEM_SHARED((NPAD, H), jnp.float32),
      sem=pltpu.SemaphoreType.DMA,
      sem2=pltpu.SemaphoreType.DMA,
  )
  n_rep = rows_sub // CHUNK

  def body(x_hbm, src_hbm, dst_hbm, zrow_hbm, iota_hbm, acc_out,
           *, idx_s, idx_d, rows, rows2, izb, acc_sh, sem, sem2):
    cid = lax.axis_index("c")
    sid = lax.axis_index("s")
    wid = cid * NS + sid

    pltpu.sync_copy(iota_hbm.at[sid], izb)
    pltpu.sync_copy(zrow_hbm, rows)
    for r in range(n_rep):
      pltpu.sync_copy(rows, acc_sh.at[izb.at[r]])
    plsc.subcore_barrier()

    bufs = (rows, rows2)
    sems = (sem, sem2)

    def slab(s, carry):
      pltpu.sync_copy(src_hbm.at[wid * slabs + s], idx_s)
      pltpu.sync_copy(dst_hbm.at[wid * slabs + s], idx_d)
      pltpu.async_copy(x_hbm.at[idx_s.at[0]], bufs[0], sems[0])
      for c in range(SLABCH):
        pltpu.make_async_copy(x_hbm, bufs[c % 2], sems[c % 2]).wait()
        if c + 1 < SLABCH:
          pltpu.async_copy(
              x_hbm.at[idx_s.at[c + 1]], bufs[(c + 1) % 2], sems[(c + 1) % 2])
        pltpu.sync_copy(bufs[c % 2], acc_sh.at[idx_d.at[c]], add=True)
      return carry
    lax.fori_loop(0, slabs, slab, 0)

    plsc.subcore_barrier()
    for r in range(n_rep):
      out_sl = pl.ds(cid * NPAD + sid * rows_sub + r * CHUNK, CHUNK)
      pltpu.async_copy(acc_sh.at[izb.at[r]], rows, sem).wait()
      pltpu.sync_copy(rows, acc_out.at[out_sl])

  return pl.kernel(
      body,
      out_type=jax.ShapeDtypeStruct((NC * NPAD, H), jnp.float32),
      mesh=_MESH,
      scratch_types=scratch,
  )


def _make_gather(n_out, nchunks):
  scratch = dict(
      idx_v=pltpu.VMEM((nchunks, GCHUNK), jnp.int32),
      rows=pltpu.VMEM((GCHUNK, H), jnp.float32),
      rows2=pltpu.VMEM((GCHUNK, H), jnp.float32),
      sem=pltpu.SemaphoreType.DMA,
      sem2=pltpu.SemaphoreType.DMA,
  )

  def body(table_hbm, idx_hbm, out_hbm, *, idx_v, rows, rows2, sem, sem2):
    cid = lax.axis_index("c")
    sid = lax.axis_index("s")
    wid = cid * NS + sid
    base = wid * nchunks * GCHUNK
    pltpu.sync_copy(idx_hbm.at[wid], idx_v)

    bufs = (rows, rows2)
    sems = (sem, sem2)
    pltpu.async_copy(table_hbm.at[idx_v.at[0]], bufs[0], sems[0])
    for c in range(nchunks):
      pltpu.make_async_copy(table_hbm, bufs[c % 2], sems[c % 2]).wait()
      if c + 1 < nchunks:
        pltpu.async_copy(
            table_hbm.at[idx_v.at[c + 1]], bufs[(c + 1) % 2], sems[(c + 1) % 2])
      pltpu.sync_copy(bufs[c % 2], out_hbm.at[pl.ds(base + c * GCHUNK, GCHUNK)])

  return pl.kernel(
      body,
      out_type=jax.ShapeDtypeStruct((n_out, H), jnp.float32),
      mesh=_MESH,
      scratch_types=scratch,
  )


def _dot_t(a, w):
  return lax.dot_general(a, w, (((1,), (1,)), ((), ())),
                         preferred_element_type=jnp.float32)


def _sage_combine(acc, cnt, x_dst, wl, wr, b):
  n = x_dst.shape[0]
  bs = 1000
  acc = acc.reshape(NC, NPAD, H)

  def body(a_ref, c_ref, xd_ref, wl_ref, wr_ref, b_ref, o_ref):
    p = a_ref[0] + a_ref[1]
    c = c_ref[0, :, 0:1] + c_ref[1, :, 0:1]
    agg = p / jnp.maximum(c, 1.0)
    y = _dot_t(agg, wl_ref[...]) + b_ref[...] + _dot_t(xd_ref[...], wr_ref[...])
    o_ref[...] = jnp.maximum(y, 0.0)

  return pl.pallas_call(
      body,
      grid=(n // bs,),
      in_specs=[
          pl.BlockSpec((NC, bs, H), lambda i: (0, i, 0)),
          pl.BlockSpec((NC, bs, CW), lambda i: (0, i, 0)),
          pl.BlockSpec((bs, H), lambda i: (i, 0)),
          pl.BlockSpec((H, H), lambda i: (0, 0)),
          pl.BlockSpec((H, H), lambda i: (0, 0)),
          pl.BlockSpec((1, H), lambda i: (0, 0)),
      ],
      out_specs=pl.BlockSpec((bs, H), lambda i: (i, 0)),
      out_shape=jax.ShapeDtypeStruct((n, H), jnp.float32),
  )(acc, cnt, x_dst, wl, wr, b)


def _linear(x, w, b):
  n = x.shape[0]
  o = w.shape[0]
  bs = 1000

  def body(x_ref, w_ref, b_ref, o_ref):
    o_ref[...] = _dot_t(x_ref[...], w_ref[...]) + b_ref[...]

  return pl.pallas_call(
      body,
      grid=(n // bs,),
      in_specs=[
          pl.BlockSpec((bs, H), lambda i: (i, 0)),
          pl.BlockSpec((o, H), lambda i: (0, 0)),
          pl.BlockSpec((1, o), lambda i: (0, 0)),
      ],
      out_specs=pl.BlockSpec((bs, o), lambda i: (i, 0)),
      out_shape=jax.ShapeDtypeStruct((n, o), jnp.float32),
  )(x, w, b)


def _decoder_final(g, w2, b2):
  bs = 800

  def body(gc_ref, gp_ref, w2_ref, b2_ref, o_ref):
    h = jnp.maximum(gc_ref[...] + gp_ref[...], 0.0)
    y = lax.dot_general(h, w2_ref[...], (((1,), (0,)), ((), ())),
                        preferred_element_type=jnp.float32)
    o_ref[...] = y[:, 0:1] + b2_ref[0, 0]

  return pl.pallas_call(
      body,
      grid=(N_LBL // bs,),
      in_specs=[
          pl.BlockSpec((bs, H), lambda i: (i, 0)),
          pl.BlockSpec((bs, H), lambda i: (i + HALF // bs, 0)),
          pl.BlockSpec((H, H), lambda i: (0, 0)),
          pl.BlockSpec((1, 1), lambda i: (0, 0)),
      ],
      out_specs=pl.BlockSpec((bs, 1), lambda i: (i, 0)),
      out_shape=jax.ShapeDtypeStruct((N_LBL, 1), jnp.float32),
  )(g, g, jnp.broadcast_to(w2.T, (H, H)), b2)


_GN_CH = 2 * HALF // NW // GCHUNK

_segsum = _make_segsum(SLABS)
_segsum2 = _make_segsum(2 * SLABS)
_gather = _make_gather(2 * HALF, _GN_CH)


def _pad_flat(ei, n_dst):
  e = ei.shape[1]
  src = jnp.concatenate([ei[0], jnp.zeros((E_PAD - e,), jnp.int32)])
  dst = jnp.concatenate([ei[1], jnp.full((E_PAD - e,), n_dst, jnp.int32)])
  return src, dst


def _slabbed(a, slabs):
  return a.reshape(NW * slabs, SLABCH, CHUNK)


@jax.jit
def kernel(x_product, x_customer, edge_index_pp, edge_index_pc,
           edge_label_index, Wl_i1, Wr_i1, Wl_i2, Wr_i2, W_ilin, Wl_u1, Wr_u1,
           Wl_u2, Wr_u2, Wl_u3, Wr_u3, W_ulin, W_d1, W_d2, bl_i1, bl_i2,
           b_ilin, bl_u1, bl_u2, bl_u3, b_ulin, b_d1, b_d2):
  src_pp, dst_pp = _pad_flat(edge_index_pp, NPROD)
  src_pc, dst_pc = _pad_flat(edge_index_pc, NCUST)

  zrow = jnp.zeros((CHUNK, H), jnp.float32)
  iota = jnp.arange(NPAD, dtype=jnp.int32).reshape(
      NS, NPAD // NS // CHUNK, CHUNK)

  acc_pp = _segsum(x_product, _slabbed(src_pp, SLABS),
                   _slabbed(dst_pp, SLABS), zrow, iota)
  acc_pcx = _segsum(x_product, _slabbed(src_pc, SLABS),
                    _slabbed(dst_pc, SLABS), zrow, iota)

  ones8 = (jnp.arange(H, dtype=jnp.int32)[None, :] // CW
           == jnp.arange(8, dtype=jnp.int32)[:, None]).astype(jnp.float32)
  rep = 1024
  ones8 = jnp.tile(ones8, (rep, 1))
  spread = 8 * (jnp.arange(2 * E_PAD, dtype=jnp.int32) % rep)
  csrc = jnp.concatenate([dst_pp % 8, dst_pc % 8]) + spread
  cdst = jnp.concatenate([dst_pp // 8, CNT_OFF + dst_pc // 8])
  co = _segsum2(ones8, _slabbed(csrc, 2 * SLABS),
                _slabbed(cdst, 2 * SLABS), zrow, iota)
  co = co.reshape(NC, NPAD, 8, CW)
  cnt_pp = co[:, :NPROD // 8].reshape(NC, NPROD, CW)
  cnt_pc = co[:, CNT_OFF:CNT_OFF + NCUST // 8].reshape(NC, NCUST, CW)

  b2d = lambda b: b.reshape(1, -1)
  px = _sage_combine(acc_pp, cnt_pp, x_product, Wl_u1, Wr_u1, b2d(bl_u1))
  ix1 = _sage_combine(acc_pp, cnt_pp, x_product, Wl_i1, Wr_i1, b2d(bl_i1))
  cx1 = _sage_combine(acc_pcx, cnt_pc, x_customer, Wl_u2, Wr_u2, b2d(bl_u2))

  acc_pcpx = _segsum(px, _slabbed(src_pc, SLABS),
                     _slabbed(dst_pc, SLABS), zrow, iota)
  acc_ppix = _segsum(ix1, _slabbed(src_pp, SLABS),
                     _slabbed(dst_pp, SLABS), zrow, iota)

  cx2 = _sage_combine(acc_pcpx, cnt_pc, cx1, Wl_u3, Wr_u3, b2d(bl_u3))
  ix2 = _sage_combine(acc_ppix, cnt_pp, ix1, Wl_i2, Wr_i2, b2d(bl_i2))
  zc = _linear(cx2, W_ulin, b2d(b_ulin))
  zp = _linear(ix2, W_ilin, b2d(b_ilin))
  uc = _linear(zc, W_d1[:, :H], b2d(b_d1))
  up = _linear(zp, W_d1[:, H:], jnp.zeros((1, H), jnp.float32))

  table = jnp.concatenate([uc, up], axis=0)
  pad = jnp.zeros((HALF - N_LBL,), jnp.int32)
  gidx = jnp.concatenate(
      [edge_label_index[0], pad, edge_label_index[1] + NCUST, pad], axis=0)
  g = _gather(table, gidx.reshape(NW, _GN_CH, GCHUNK))

  return _decoder_final(g, W_d2, b2d(b_d2))


# speedup vs baseline: 2.4391x; 1.0932x over previous
import jax
import jax.numpy as jnp
from jax import lax
from jax.experimental import pallas as pl
from jax.experimental.pallas import tpu as pltpu
from jax.experimental.pallas import tpu_sc as plsc

H = 128
NPROD = 10000
NCUST = 10000
N_LBL = 100000

NC = 2
NS = 16
NW = NC * NS
CHUNK = 128
CW = 16
NPAD = 10240
SLABS = 8
SLABCH = 10
E_PAD = NW * SLABS * SLABCH * CHUNK
CNT_OFF = 1256
GCHUNK = 128
HALF = 102400

_MESH = plsc.VectorSubcoreMesh(core_axis_name="c", subcore_axis_name="s")


def _make_segsum(slabs):
  rows_sub = NPAD // NS

  scratch = dict(
      idx_s=pltpu.VMEM((SLABCH, CHUNK), jnp.int32),
      idx_d=pltpu.VMEM((SLABCH, CHUNK), jnp.int32),
      rows=pltpu.VMEM((CHUNK, H), jnp.float32),
      rows2=pltpu.VMEM((CHUNK, H), jnp.float32),
      izb=pltpu.VMEM((NPAD // NS // CHUNK, CHUNK), jnp.int32),
      acc_sh=pltpu.VMEM_SHARED((NPAD, H), jnp.float32),
      sem=pltpu.SemaphoreType.DMA,
      sem2=pltpu.SemaphoreType.DMA,
  )
  n_rep = rows_sub // CHUNK

  def body(x_hbm, src_hbm, dst_hbm, zrow_hbm, iota_hbm, acc_out,
           *, idx_s, idx_d, rows, rows2, izb, acc_sh, sem, sem2):
    cid = lax.axis_index("c")
    sid = lax.axis_index("s")
    wid = cid * NS + sid

    pltpu.sync_copy(iota_hbm.at[sid], izb)
    pltpu.sync_copy(zrow_hbm, rows)
    for r in range(n_rep):
      pltpu.sync_copy(rows, acc_sh.at[izb.at[r]])
    plsc.subcore_barrier()

    bufs = (rows, rows2)
    sems = (sem, sem2)

    def slab(s, carry):
      pltpu.sync_copy(src_hbm.at[wid * slabs + s], idx_s)
      pltpu.sync_copy(dst_hbm.at[wid * slabs + s], idx_d)
      pltpu.async_copy(x_hbm.at[idx_s.at[0]], bufs[0], sems[0])
      for c in range(SLABCH):
        pltpu.make_async_copy(x_hbm, bufs[c % 2], sems[c % 2]).wait()
        if c + 1 < SLABCH:
          pltpu.async_copy(
              x_hbm.at[idx_s.at[c + 1]], bufs[(c + 1) % 2], sems[(c + 1) % 2])
        pltpu.sync_copy(bufs[c % 2], acc_sh.at[idx_d.at[c]], add=True)
      return carry
    lax.fori_loop(0, slabs, slab, 0)

    plsc.subcore_barrier()
    for r in range(n_rep):
      out_sl = pl.ds(cid * NPAD + sid * rows_sub + r * CHUNK, CHUNK)
      pltpu.async_copy(acc_sh.at[izb.at[r]], rows, sem).wait()
      pltpu.sync_copy(rows, acc_out.at[out_sl])

  return pl.kernel(
      body,
      out_type=jax.ShapeDtypeStruct((NC * NPAD, H), jnp.float32),
      mesh=_MESH,
      scratch_types=scratch,
  )


def _make_gather(n_out, nchunks):
  scratch = dict(
      idx_v=pltpu.VMEM((nchunks, GCHUNK), jnp.int32),
      rows=pltpu.VMEM((GCHUNK, H), jnp.float32),
      rows2=pltpu.VMEM((GCHUNK, H), jnp.float32),
      sem=pltpu.SemaphoreType.DMA,
      sem2=pltpu.SemaphoreType.DMA,
  )

  def body(table_hbm, idx_hbm, out_hbm, *, idx_v, rows, rows2, sem, sem2):
    cid = lax.axis_index("c")
    sid = lax.axis_index("s")
    wid = cid * NS + sid
    base = wid * nchunks * GCHUNK
    pltpu.sync_copy(idx_hbm.at[wid], idx_v)

    bufs = (rows, rows2)
    sems = (sem, sem2)
    pltpu.async_copy(table_hbm.at[idx_v.at[0]], bufs[0], sems[0])
    for c in range(nchunks):
      pltpu.make_async_copy(table_hbm, bufs[c % 2], sems[c % 2]).wait()
      if c + 1 < nchunks:
        pltpu.async_copy(
            table_hbm.at[idx_v.at[c + 1]], bufs[(c + 1) % 2], sems[(c + 1) % 2])
      pltpu.sync_copy(bufs[c % 2], out_hbm.at[pl.ds(base + c * GCHUNK, GCHUNK)])

  return pl.kernel(
      body,
      out_type=jax.ShapeDtypeStruct((n_out, H), jnp.float32),
      mesh=_MESH,
      scratch_types=scratch,
  )


def _dot_t(a, w):
  return lax.dot_general(a, w, (((1,), (1,)), ((), ())),
                         preferred_element_type=jnp.float32)


def _sage_combine(acc, cnt, x_dst, wl, wr, b):
  n = x_dst.shape[0]
  bs = 1000
  acc = acc.reshape(NC, NPAD, H)

  def body(a_ref, c_ref, xd_ref, wl_ref, wr_ref, b_ref, o_ref):
    p = a_ref[0] + a_ref[1]
    c = c_ref[0, :, 0:1] + c_ref[1, :, 0:1]
    agg = p / jnp.maximum(c, 1.0)
    y = _dot_t(agg, wl_ref[...]) + b_ref[...] + _dot_t(xd_ref[...], wr_ref[...])
    o_ref[...] = jnp.maximum(y, 0.0)

  return pl.pallas_call(
      body,
      grid=(n // bs,),
      in_specs=[
          pl.BlockSpec((NC, bs, H), lambda i: (0, i, 0)),
          pl.BlockSpec((NC, bs, CW), lambda i: (0, i, 0)),
          pl.BlockSpec((bs, H), lambda i: (i, 0)),
          pl.BlockSpec((H, H), lambda i: (0, 0)),
          pl.BlockSpec((H, H), lambda i: (0, 0)),
          pl.BlockSpec((1, H), lambda i: (0, 0)),
      ],
      out_specs=pl.BlockSpec((bs, H), lambda i: (i, 0)),
      out_shape=jax.ShapeDtypeStruct((n, H), jnp.float32),
  )(acc, cnt, x_dst, wl, wr, b)


def _linear(x, w, b):
  n = x.shape[0]
  o = w.shape[0]
  bs = 1000

  def body(x_ref, w_ref, b_ref, o_ref):
    o_ref[...] = _dot_t(x_ref[...], w_ref[...]) + b_ref[...]

  return pl.pallas_call(
      body,
      grid=(n // bs,),
      in_specs=[
          pl.BlockSpec((bs, H), lambda i: (i, 0)),
          pl.BlockSpec((o, H), lambda i: (0, 0)),
          pl.BlockSpec((1, o), lambda i: (0, 0)),
      ],
      out_specs=pl.BlockSpec((bs, o), lambda i: (i, 0)),
      out_shape=jax.ShapeDtypeStruct((n, o), jnp.float32),
  )(x, w, b)


def _decoder_final(g, w2, b2):
  bs = 800

  def body(gc_ref, gp_ref, w2_ref, b2_ref, o_ref):
    h = jnp.maximum(gc_ref[...] + gp_ref[...], 0.0)
    y = lax.dot_general(h, w2_ref[...], (((1,), (0,)), ((), ())),
                        preferred_element_type=jnp.float32)
    o_ref[...] = y[:, 0:1] + b2_ref[0, 0]

  return pl.pallas_call(
      body,
      grid=(N_LBL // bs,),
      in_specs=[
          pl.BlockSpec((bs, H), lambda i: (i, 0)),
          pl.BlockSpec((bs, H), lambda i: (i + HALF // bs, 0)),
          pl.BlockSpec((H, H), lambda i: (0, 0)),
          pl.BlockSpec((1, 1), lambda i: (0, 0)),
      ],
      out_specs=pl.BlockSpec((bs, 1), lambda i: (i, 0)),
      out_shape=jax.ShapeDtypeStruct((N_LBL, 1), jnp.float32),
  )(g, g, jnp.broadcast_to(w2.T, (H, H)), b2)


_GN_CH = 2 * HALF // NW // GCHUNK

_segsum = _make_segsum(SLABS)
_segsum2 = _make_segsum(2 * SLABS)
_gather = _make_gather(2 * HALF, _GN_CH)


def _pad_flat(ei, n_dst):
  e = ei.shape[1]
  src = jnp.concatenate([ei[0], jnp.zeros((E_PAD - e,), jnp.int32)])
  dst = jnp.concatenate([ei[1], jnp.full((E_PAD - e,), n_dst, jnp.int32)])
  return src, dst


def _slabbed(a, slabs):
  return a.reshape(NW * slabs, SLABCH, CHUNK)


@jax.jit
def kernel(x_product, x_customer, edge_index_pp, edge_index_pc,
           edge_label_index, Wl_i1, Wr_i1, Wl_i2, Wr_i2, W_ilin, Wl_u1, Wr_u1,
           Wl_u2, Wr_u2, Wl_u3, Wr_u3, W_ulin, W_d1, W_d2, bl_i1, bl_i2,
           b_ilin, bl_u1, bl_u2, bl_u3, b_ulin, b_d1, b_d2):
  src_pp, dst_pp = _pad_flat(edge_index_pp, NPROD)
  src_pc, dst_pc = _pad_flat(edge_index_pc, NCUST)

  zrow = jnp.zeros((CHUNK, H), jnp.float32)
  iota = jnp.arange(NPAD, dtype=jnp.int32).reshape(
      NS, NPAD // NS // CHUNK, CHUNK)

  acc_pp = _segsum(x_product, _slabbed(src_pp, SLABS),
                   _slabbed(dst_pp, SLABS), zrow, iota)
  acc_pcx = _segsum(x_product, _slabbed(src_pc, SLABS),
                    _slabbed(dst_pc, SLABS), zrow, iota)

  ones8 = (jnp.arange(H, dtype=jnp.int32)[None, :] // CW
           == jnp.arange(8, dtype=jnp.int32)[:, None]).astype(jnp.float32)
  rep = 1024
  ones8 = jnp.tile(ones8, (rep, 1))
  spread = 8 * (jnp.arange(2 * E_PAD, dtype=jnp.int32) % rep)
  csrc = jnp.concatenate([dst_pp % 8, dst_pc % 8]) + spread
  cdst = jnp.concatenate([dst_pp // 8, CNT_OFF + dst_pc // 8])
  co = _segsum2(ones8, _slabbed(csrc, 2 * SLABS),
                _slabbed(cdst, 2 * SLABS), zrow, iota)
  co = co.reshape(NC, NPAD, 8, CW)
  cnt_pp = co[:, :NPROD // 8].reshape(NC, NPROD, CW)
  cnt_pc = co[:, CNT_OFF:CNT_OFF + NCUST // 8].reshape(NC, NCUST, CW)

  b2d = lambda b: b.reshape(1, -1)
  px = _sage_combine(acc_pp, cnt_pp, x_product, Wl_u1, Wr_u1, b2d(bl_u1))
  ix1 = _sage_combine(acc_pp, cnt_pp, x_product, Wl_i1, Wr_i1, b2d(bl_i1))
  cx1 = _sage_combine(acc_pcx, cnt_pc, x_customer, Wl_u2, Wr_u2, b2d(bl_u2))

  acc_pcpx = _segsum(px, _slabbed(src_pc, SLABS),
                     _slabbed(dst_pc, SLABS), zrow, iota)
  acc_ppix = _segsum(ix1, _slabbed(src_pp, SLABS),
                     _slabbed(dst_pp, SLABS), zrow, iota)

  cx2 = _sage_combine(acc_pcpx, cnt_pc, cx1, Wl_u3, Wr_u3, b2d(bl_u3))
  ix2 = _sage_combine(acc_ppix, cnt_pp, ix1, Wl_i2, Wr_i2, b2d(bl_i2))
  zc = _linear(cx2, W_ulin, b2d(b_ulin))
  zp = _linear(ix2, W_ilin, b2d(b_ilin))
  uc = _linear(zc, W_d1[:, :H], b2d(b_d1))
  up = _linear(zp, W_d1[:, H:], jnp.zeros((1, H), jnp.float32))

  table = jnp.concatenate([uc, up], axis=0)
  pad = jnp.zeros((HALF - N_LBL,), jnp.int32)
  gidx = jnp.concatenate(
      [edge_label_index[0], pad, edge_label_index[1] + NCUST, pad], axis=0)
  g = _gather(table, gidx.reshape(NW, _GN_CH, GCHUNK))

  return _decoder_final(g, W_d2, b2d(b_d2))


# speedup vs baseline: 2.7500x; 1.1275x over previous
import jax
import jax.numpy as jnp
from jax import lax
from jax.experimental import pallas as pl
from jax.experimental.pallas import tpu as pltpu
from jax.experimental.pallas import tpu_sc as plsc

H = 128
NPROD = 10000
NCUST = 10000
N_LBL = 100000

NC = 2
NS = 16
NW = NC * NS
CHUNK = 128
CW = 16
NPAD = 10240
SLABS = 8
SLABCH = 10
E_PAD = NW * SLABS * SLABCH * CHUNK
CNT_OFF = 1256
GCHUNK = 128
HALF = 102400

_MESH = plsc.VectorSubcoreMesh(core_axis_name="c", subcore_axis_name="s")


def _make_segsum(slabs):
  rows_sub = NPAD // NS

  scratch = dict(
      idx_s=pltpu.VMEM((SLABCH, CHUNK), jnp.int32),
      idx_d=pltpu.VMEM((SLABCH, CHUNK), jnp.int32),
      rows=pltpu.VMEM((CHUNK, H), jnp.float32),
      rows2=pltpu.VMEM((CHUNK, H), jnp.float32),
      izb=pltpu.VMEM((NPAD // NS // CHUNK, CHUNK), jnp.int32),
      acc_sh=pltpu.VMEM_SHARED((NPAD, H), jnp.float32),
      sem=pltpu.SemaphoreType.DMA,
      sem2=pltpu.SemaphoreType.DMA,
  )
  n_rep = rows_sub // CHUNK

  def body(x_hbm, src_hbm, dst_hbm, zrow_hbm, iota_hbm, acc_out,
           *, idx_s, idx_d, rows, rows2, izb, acc_sh, sem, sem2):
    cid = lax.axis_index("c")
    sid = lax.axis_index("s")
    wid = cid * NS + sid

    pltpu.sync_copy(iota_hbm.at[sid], izb)
    pltpu.sync_copy(zrow_hbm, rows)
    for r in range(n_rep):
      pltpu.sync_copy(rows, acc_sh.at[izb.at[r]])
    plsc.subcore_barrier()

    bufs = (rows, rows2)
    sems = (sem, sem2)

    def slab(s, carry):
      pltpu.sync_copy(src_hbm.at[wid * slabs + s], idx_s)
      pltpu.sync_copy(dst_hbm.at[wid * slabs + s], idx_d)
      pltpu.async_copy(x_hbm.at[idx_s.at[0]], bufs[0], sems[0])
      for c in range(SLABCH):
        pltpu.make_async_copy(x_hbm, bufs[c % 2], sems[c % 2]).wait()
        if c + 1 < SLABCH:
          pltpu.async_copy(
              x_hbm.at[idx_s.at[c + 1]], bufs[(c + 1) % 2], sems[(c + 1) % 2])
        pltpu.sync_copy(bufs[c % 2], acc_sh.at[idx_d.at[c]], add=True)
      return carry
    lax.fori_loop(0, slabs, slab, 0)

    plsc.subcore_barrier()
    for r in range(n_rep):
      out_sl = pl.ds(cid * NPAD + sid * rows_sub + r * CHUNK, CHUNK)
      pltpu.async_copy(acc_sh.at[izb.at[r]], rows, sem).wait()
      pltpu.sync_copy(rows, acc_out.at[out_sl])

  return pl.kernel(
      body,
      out_type=jax.ShapeDtypeStruct((NC * NPAD, H), jnp.float32),
      mesh=_MESH,
      scratch_types=scratch,
  )


def _make_gather(n_out, nchunks):
  scratch = dict(
      idx_v=pltpu.VMEM((nchunks, GCHUNK), jnp.int32),
      rows=pltpu.VMEM((GCHUNK, H), jnp.float32),
      rows2=pltpu.VMEM((GCHUNK, H), jnp.float32),
      sem=pltpu.SemaphoreType.DMA,
      sem2=pltpu.SemaphoreType.DMA,
  )

  def body(table_hbm, idx_hbm, out_hbm, *, idx_v, rows, rows2, sem, sem2):
    cid = lax.axis_index("c")
    sid = lax.axis_index("s")
    wid = cid * NS + sid
    base = wid * nchunks * GCHUNK
    pltpu.sync_copy(idx_hbm.at[wid], idx_v)

    bufs = (rows, rows2)
    sems = (sem, sem2)
    pltpu.async_copy(table_hbm.at[idx_v.at[0]], bufs[0], sems[0])
    for c in range(nchunks):
      pltpu.make_async_copy(table_hbm, bufs[c % 2], sems[c % 2]).wait()
      if c + 1 < nchunks:
        pltpu.async_copy(
            table_hbm.at[idx_v.at[c + 1]], bufs[(c + 1) % 2], sems[(c + 1) % 2])
      pltpu.sync_copy(bufs[c % 2], out_hbm.at[pl.ds(base + c * GCHUNK, GCHUNK)])

  return pl.kernel(
      body,
      out_type=jax.ShapeDtypeStruct((n_out, H), jnp.float32),
      mesh=_MESH,
      scratch_types=scratch,
  )


def _dot_t(a, w):
  return lax.dot_general(a, w, (((1,), (1,)), ((), ())),
                         preferred_element_type=jnp.float32)


def _sage_combine(acc, cnt, x_dst, wl, wr, b):
  n = x_dst.shape[0]
  bs = 1000
  acc = acc.reshape(NC, NPAD, H)

  def body(a_ref, c_ref, xd_ref, wl_ref, wr_ref, b_ref, o_ref):
    p = a_ref[0] + a_ref[1]
    c = c_ref[0, :, 0:1] + c_ref[1, :, 0:1]
    agg = p / jnp.maximum(c, 1.0)
    y = _dot_t(agg, wl_ref[...]) + b_ref[...] + _dot_t(xd_ref[...], wr_ref[...])
    o_ref[...] = jnp.maximum(y, 0.0)

  return pl.pallas_call(
      body,
      grid=(n // bs,),
      in_specs=[
          pl.BlockSpec((NC, bs, H), lambda i: (0, i, 0)),
          pl.BlockSpec((NC, bs, CW), lambda i: (0, i, 0)),
          pl.BlockSpec((bs, H), lambda i: (i, 0)),
          pl.BlockSpec((H, H), lambda i: (0, 0)),
          pl.BlockSpec((H, H), lambda i: (0, 0)),
          pl.BlockSpec((1, H), lambda i: (0, 0)),
      ],
      out_specs=pl.BlockSpec((bs, H), lambda i: (i, 0)),
      out_shape=jax.ShapeDtypeStruct((n, H), jnp.float32),
  )(acc, cnt, x_dst, wl, wr, b)


def _sage_combine_full(acc, cnt, x_dst, wl, wr, b):
  n = x_dst.shape[0]
  bs = 1000

  def body(a_ref, c_ref, xd_ref, wl_ref, wr_ref, b_ref, o_ref):
    c = c_ref[0, :, 0:1] + c_ref[1, :, 0:1]
    agg = a_ref[...] / jnp.maximum(c, 1.0)
    y = _dot_t(agg, wl_ref[...]) + b_ref[...] + _dot_t(xd_ref[...], wr_ref[...])
    o_ref[...] = jnp.maximum(y, 0.0)

  return pl.pallas_call(
      body,
      grid=(n // bs,),
      in_specs=[
          pl.BlockSpec((bs, H), lambda i: (i, 0)),
          pl.BlockSpec((NC, bs, CW), lambda i: (0, i, 0)),
          pl.BlockSpec((bs, H), lambda i: (i, 0)),
          pl.BlockSpec((H, H), lambda i: (0, 0)),
          pl.BlockSpec((H, H), lambda i: (0, 0)),
          pl.BlockSpec((1, H), lambda i: (0, 0)),
      ],
      out_specs=pl.BlockSpec((bs, H), lambda i: (i, 0)),
      out_shape=jax.ShapeDtypeStruct((n, H), jnp.float32),
  )(acc, cnt, x_dst, wl, wr, b)


def _linear(x, w, b):
  n = x.shape[0]
  o = w.shape[0]
  bs = 1000

  def body(x_ref, w_ref, b_ref, o_ref):
    o_ref[...] = _dot_t(x_ref[...], w_ref[...]) + b_ref[...]

  return pl.pallas_call(
      body,
      grid=(n // bs,),
      in_specs=[
          pl.BlockSpec((bs, H), lambda i: (i, 0)),
          pl.BlockSpec((o, H), lambda i: (0, 0)),
          pl.BlockSpec((1, o), lambda i: (0, 0)),
      ],
      out_specs=pl.BlockSpec((bs, o), lambda i: (i, 0)),
      out_shape=jax.ShapeDtypeStruct((n, o), jnp.float32),
  )(x, w, b)


def _decoder_final(g, w2, b2):
  bs = 800

  def body(gc_ref, gp_ref, w2_ref, b2_ref, o_ref):
    h = jnp.maximum(gc_ref[...] + gp_ref[...], 0.0)
    y = lax.dot_general(h, w2_ref[...], (((1,), (0,)), ((), ())),
                        preferred_element_type=jnp.float32)
    o_ref[...] = y[:, 0:1] + b2_ref[0, 0]

  return pl.pallas_call(
      body,
      grid=(N_LBL // bs,),
      in_specs=[
          pl.BlockSpec((bs, H), lambda i: (i, 0)),
          pl.BlockSpec((bs, H), lambda i: (i + HALF // bs, 0)),
          pl.BlockSpec((H, H), lambda i: (0, 0)),
          pl.BlockSpec((1, 1), lambda i: (0, 0)),
      ],
      out_specs=pl.BlockSpec((bs, 1), lambda i: (i, 0)),
      out_shape=jax.ShapeDtypeStruct((N_LBL, 1), jnp.float32),
  )(g, g, jnp.broadcast_to(w2.T, (H, H)), b2)


_GN_CH = 2 * HALF // NW // GCHUNK

_segsum2 = _make_segsum(2 * SLABS)
_gather = _make_gather(2 * HALF, _GN_CH)


def _pad_flat(ei, n_dst):
  e = ei.shape[1]
  src = jnp.concatenate([ei[0], jnp.zeros((E_PAD - e,), jnp.int32)])
  dst = jnp.concatenate([ei[1], jnp.full((E_PAD - e,), n_dst, jnp.int32)])
  return src, dst


def _slabbed(a, slabs):
  return a.reshape(NW * slabs, SLABCH, CHUNK)


@jax.jit
def kernel(x_product, x_customer, edge_index_pp, edge_index_pc,
           edge_label_index, Wl_i1, Wr_i1, Wl_i2, Wr_i2, W_ilin, Wl_u1, Wr_u1,
           Wl_u2, Wr_u2, Wl_u3, Wr_u3, W_ulin, W_d1, W_d2, bl_i1, bl_i2,
           b_ilin, bl_u1, bl_u2, bl_u3, b_ulin, b_d1, b_d2):
  src_pp, dst_pp = _pad_flat(edge_index_pp, NPROD)
  src_pc, dst_pc = _pad_flat(edge_index_pc, NCUST)

  zrow = jnp.zeros((CHUNK, H), jnp.float32)
  iota = jnp.arange(NPAD, dtype=jnp.int32).reshape(
      NS, NPAD // NS // CHUNK, CHUNK)

  src12 = jnp.concatenate([src_pp, src_pc])
  dst12 = jnp.concatenate([dst_pp, dst_pc])
  a12 = _segsum2(x_product, _slabbed(src12, 2 * SLABS),
                 _slabbed(dst12, 2 * SLABS), zrow, iota)
  acc_pp, acc_pcx = a12[:NPAD], a12[NPAD:]

  ones8 = (jnp.arange(H, dtype=jnp.int32)[None, :] // CW
           == jnp.arange(8, dtype=jnp.int32)[:, None]).astype(jnp.float32)
  rep = 1024
  ones8 = jnp.tile(ones8, (rep, 1))
  spread = 8 * (jnp.arange(2 * E_PAD, dtype=jnp.int32) % rep)
  csrc = jnp.concatenate([dst_pp % 8, dst_pc % 8]) + spread
  cdst = jnp.concatenate([dst_pp // 8, CNT_OFF + dst_pc // 8])
  co = _segsum2(ones8, _slabbed(csrc, 2 * SLABS),
                _slabbed(cdst, 2 * SLABS), zrow, iota)
  co = co.reshape(NC, NPAD, 8, CW)
  cnt_pp = co[:, :NPROD // 8].reshape(NC, NPROD, CW)
  cnt_pc = co[:, CNT_OFF:CNT_OFF + NCUST // 8].reshape(NC, NCUST, CW)

  b2d = lambda b: b.reshape(1, -1)
  px = _sage_combine_full(acc_pp, cnt_pp, x_product, Wl_u1, Wr_u1,
                          b2d(bl_u1))
  ix1 = _sage_combine_full(acc_pp, cnt_pp, x_product, Wl_i1, Wr_i1,
                           b2d(bl_i1))
  cx1 = _sage_combine_full(acc_pcx, cnt_pc, x_customer, Wl_u2, Wr_u2,
                           b2d(bl_u2))

  xs34 = jnp.concatenate([px, ix1], axis=0)
  src34 = jnp.concatenate([src_pc, src_pp + NPROD])
  dst34 = jnp.concatenate([dst_pc, dst_pp])
  a34 = _segsum2(xs34, _slabbed(src34, 2 * SLABS),
                 _slabbed(dst34, 2 * SLABS), zrow, iota)
  acc_pcpx, acc_ppix = a34[:NPAD], a34[NPAD:]

  cx2 = _sage_combine_full(acc_pcpx, cnt_pc, cx1, Wl_u3, Wr_u3, b2d(bl_u3))
  ix2 = _sage_combine_full(acc_ppix, cnt_pp, ix1, Wl_i2, Wr_i2, b2d(bl_i2))
  zc = _linear(cx2, W_ulin, b2d(b_ulin))
  zp = _linear(ix2, W_ilin, b2d(b_ilin))
  uc = _linear(zc, W_d1[:, :H], b2d(b_d1))
  up = _linear(zp, W_d1[:, H:], jnp.zeros((1, H), jnp.float32))

  table = jnp.concatenate([uc, up], axis=0)
  pad = jnp.zeros((HALF - N_LBL,), jnp.int32)
  gidx = jnp.concatenate(
      [edge_label_index[0], pad, edge_label_index[1] + NCUST, pad], axis=0)
  g = _gather(table, gidx.reshape(NW, _GN_CH, GCHUNK))

  return _decoder_final(g, W_d2, b2d(b_d2))
